# baseline (speedup 1.0000x reference)
import functools

import jax
import jax.numpy as jnp
from jax import lax
from jax.experimental import pallas as pl
from jax.experimental.pallas import tpu as pltpu
from jax.experimental.pallas import tpu_sc as plsc

EMB = 32
BATCH = 16384
NC = 2
NS = 16
NW = NC * NS
BPW = BATCH // NW
CHUNK = 128
NCH = BPW // CHUNK

BM = 2048


def _gather_kernel(user_hbm, movie_hbm, utab_hbm, mtab_hbm, uout_hbm, mout_hbm,
                   uidx_v, midx_v, urows_v, mrows_v, sem):
    wid = lax.axis_index("s") * NC + lax.axis_index("c")
    base = wid * BPW
    pltpu.sync_copy(user_hbm.at[wid], uidx_v)
    pltpu.sync_copy(movie_hbm.at[wid], midx_v)
    copies = []
    for j in range(NCH):
        copies.append(pltpu.async_copy(
            utab_hbm.at[uidx_v.at[j]], urows_v.at[pl.ds(j * CHUNK, CHUNK)], sem))
        copies.append(pltpu.async_copy(
            mtab_hbm.at[midx_v.at[j]], mrows_v.at[pl.ds(j * CHUNK, CHUNK)], sem))
    for c in copies:
        c.wait()
    pltpu.sync_copy(urows_v, uout_hbm.at[pl.ds(base, BPW)])
    pltpu.sync_copy(mrows_v, mout_hbm.at[pl.ds(base, BPW)])


@functools.partial(jax.jit, static_argnums=())
def _gather(user, movie, utab, mtab):
    mesh = plsc.VectorSubcoreMesh(core_axis_name="c", subcore_axis_name="s")
    k = functools.partial(
        pl.kernel,
        mesh=mesh,
        out_type=[
            jax.ShapeDtypeStruct((BATCH, EMB), jnp.float32),
            jax.ShapeDtypeStruct((BATCH, EMB), jnp.float32),
        ],
        scratch_types=[
            pltpu.VMEM((NCH, CHUNK), jnp.int32),
            pltpu.VMEM((NCH, CHUNK), jnp.int32),
            pltpu.VMEM((BPW, EMB), jnp.float32),
            pltpu.VMEM((BPW, EMB), jnp.float32),
            pltpu.SemaphoreType.DMA,
        ],
        compiler_params=pltpu.CompilerParams(use_tc_tiling_on_sc=False),
    )(_gather_kernel)
    return k(user.reshape(NW, NCH, CHUNK), movie.reshape(NW, NCH, CHUNK),
             utab, mtab)


def _mlp_kernel(u_ref, m_ref, w1u_ref, w1m_ref, b1_ref, w2_ref, b2_ref,
                w3t_ref, b3_ref, out_ref):
    h = jnp.dot(u_ref[...], w1u_ref[...], preferred_element_type=jnp.float32)
    h = h + jnp.dot(m_ref[...], w1m_ref[...],
                    preferred_element_type=jnp.float32)
    h = jnp.maximum(h + b1_ref[...], 0.0)
    h = jnp.dot(h, w2_ref[...], preferred_element_type=jnp.float32)
    h = jnp.maximum(h + b2_ref[...], 0.0)
    o = jnp.sum(h * w3t_ref[...], axis=1, keepdims=True) + b3_ref[...]
    out_ref[...] = 1.0 / (1.0 + jnp.exp(-o))


def _mlp(u, m, W1, b1, W2, b2, W3, b3):
    hid = W1.shape[1]
    h2 = W2.shape[1]
    grid = (BATCH // BM,)
    full = lambda shape: pl.BlockSpec(shape, lambda i: (0, 0))
    out = pl.pallas_call(
        _mlp_kernel,
        grid=grid,
        in_specs=[
            pl.BlockSpec((BM, EMB), lambda i: (i, 0)),
            pl.BlockSpec((BM, EMB), lambda i: (i, 0)),
            full((EMB, hid)),
            full((EMB, hid)),
            full((1, hid)),
            full((hid, h2)),
            full((1, h2)),
            full((1, h2)),
            full((1, 1)),
        ],
        out_specs=pl.BlockSpec((BM, 1), lambda i: (i, 0)),
        out_shape=jax.ShapeDtypeStruct((BATCH, 1), jnp.float32),
    )(u, m, W1[:EMB], W1[EMB:], b1.reshape(1, hid), W2,
      b2.reshape(1, h2), W3.reshape(1, h2), b3.reshape(1, 1))
    return out


def kernel(user, movie, user_emb_table, movie_emb_table, W1, b1, W2, b2, W3, b3):
    u, m = _gather(user.astype(jnp.int32), movie.astype(jnp.int32),
                   user_emb_table, movie_emb_table)
    out = _mlp(u, m, W1, b1, W2, b2, W3, b3)
    return jnp.squeeze(out, axis=-1)
